# Initial kernel scaffold; baseline (speedup 1.0000x reference)
import jax
import jax.numpy as jnp
from jax.experimental import pallas as pl


def kernel(contexts, missing_flag, cell_ids, learning_cell):
    raise NotImplementedError("write your pallas kernel here")


# speedup vs baseline: 150.5772x; 150.5772x over previous
import functools

import jax
import jax.numpy as jnp
from jax import lax
from jax.experimental import pallas as pl
from jax.experimental.pallas import tpu as pltpu
from jax.experimental.pallas import tpu_sc as plsc

P, N, W = 26, 16384, 50
E = P * N * W
NUM_WORKERS = 32
PER_WORKER = E // NUM_WORKERS
CHUNK = 16640
NCHUNK = PER_WORKER // CHUNK
LANES = 16


def _sc_fill(ctx_flat, flag_flat, ids_flat, table):
    mesh = plsc.VectorSubcoreMesh(core_axis_name="c", subcore_axis_name="s")

    @functools.partial(
        pl.kernel,
        mesh=mesh,
        out_type=jax.ShapeDtypeStruct((E,), jnp.float32),
        scratch_types=[
            pltpu.VMEM((CHUNK,), jnp.int32),
            pltpu.VMEM((CHUNK,), jnp.float32),
            pltpu.VMEM((CHUNK,), jnp.int32),
            pltpu.VMEM((CHUNK,), jnp.float32),
            pltpu.SemaphoreType.DMA,
        ],
    )
    def k(ctx_hbm, flag_hbm, ids_hbm, tab_hbm, out_hbm, idx_v, g_v, f_v, c_v, sem):
        wid = lax.axis_index("s") * 2 + lax.axis_index("c")
        base_w = wid * PER_WORKER

        def chunk_body(i, carry):
            base = base_w + i * CHUNK
            pltpu.sync_copy(ids_hbm.at[pl.ds(base, CHUNK)], idx_v)
            pltpu.async_copy(tab_hbm.at[idx_v], g_v, sem).wait()
            pltpu.sync_copy(flag_hbm.at[pl.ds(base, CHUNK)], f_v)
            pltpu.sync_copy(ctx_hbm.at[pl.ds(base, CHUNK)], c_v)

            def vec_body(j, carry2):
                s = pl.ds(j * LANES, LANES)
                c_v[s] = jnp.where(f_v[s] == 1, g_v[s], c_v[s])
                return carry2

            lax.fori_loop(0, CHUNK // LANES, vec_body, 0)
            pltpu.sync_copy(c_v, out_hbm.at[pl.ds(base, CHUNK)])
            return carry

        lax.fori_loop(0, NCHUNK, chunk_body, 0)

    return k(ctx_flat, flag_flat, ids_flat, table)


def kernel(contexts, missing_flag, cell_ids, learning_cell):
    ids = cell_ids.astype(jnp.int32).reshape(-1)
    filled = _sc_fill(
        contexts.reshape(-1), missing_flag.reshape(-1), ids, learning_cell
    )
    return filled.reshape(contexts.shape), learning_cell


# speedup vs baseline: 173.7696x; 1.1540x over previous
import functools

import jax
import jax.numpy as jnp
from jax import lax
from jax.experimental import pallas as pl
from jax.experimental.pallas import tpu as pltpu
from jax.experimental.pallas import tpu_sc as plsc

P, N, W = 26, 16384, 50
E = P * N * W
NUM_WORKERS = 32
PER_WORKER = E // NUM_WORKERS
CHUNK = 12800
NCHUNK = PER_WORKER // CHUNK
GROUPS = NCHUNK // 4
LANES = 16
UNROLL = 4


def _sc_fill(ctx_flat, flag_flat, ids_flat, table):
    mesh = plsc.VectorSubcoreMesh(core_axis_name="c", subcore_axis_name="s")

    @functools.partial(
        pl.kernel,
        mesh=mesh,
        out_type=jax.ShapeDtypeStruct((E,), jnp.float32),
        scratch_types=[pltpu.VMEM((CHUNK,), jnp.int32)] * 4
        + [pltpu.VMEM((CHUNK,), jnp.float32)] * 2
        + [pltpu.VMEM((CHUNK,), jnp.int32)] * 2
        + [pltpu.VMEM((CHUNK,), jnp.float32)] * 2
        + [pltpu.SemaphoreType.DMA] * 12,
    )
    def k(ctx_hbm, flag_hbm, ids_hbm, tab_hbm, out_hbm,
          i0, i1, i2, i3, g0, g1, f0, f1, c0, c1,
          si0, si1, si2, si3, sg0, sg1, sf0, sf1, sc0, sc1, so0, so1):
        idx_v = (i0, i1, i2, i3)
        g_v = (g0, g1)
        f_v = (f0, f1)
        c_v = (c0, c1)
        s_idx = (si0, si1, si2, si3)
        s_g = (sg0, sg1)
        s_f = (sf0, sf1)
        s_c = (sc0, sc1)
        s_o = (so0, so1)
        wid = lax.axis_index("s") * 2 + lax.axis_index("c")
        base_w = wid * PER_WORKER

        def src(i):
            return pl.ds(base_w + i * CHUNK, CHUNK)

        def start_ids(i, slot):
            pltpu.async_copy(ids_hbm.at[src(i)], idx_v[slot], s_idx[slot])

        def wait_ids(i, slot):
            pltpu.make_async_copy(
                ids_hbm.at[src(i)], idx_v[slot], s_idx[slot]).wait()

        def start_gather(slot_i, slot2):
            pltpu.async_copy(tab_hbm.at[idx_v[slot_i]], g_v[slot2], s_g[slot2])

        def wait_gather(slot_i, slot2):
            pltpu.make_async_copy(tab_hbm.at[idx_v[slot_i]], g_v[slot2],
                                  s_g[slot2]).wait()

        def start_fc(i, slot2):
            pltpu.async_copy(flag_hbm.at[src(i)], f_v[slot2], s_f[slot2])
            pltpu.async_copy(ctx_hbm.at[src(i)], c_v[slot2], s_c[slot2])

        def wait_fc(i, slot2):
            pltpu.make_async_copy(
                flag_hbm.at[src(i)], f_v[slot2], s_f[slot2]).wait()
            pltpu.make_async_copy(
                ctx_hbm.at[src(i)], c_v[slot2], s_c[slot2]).wait()

        def start_out(i, slot2):
            pltpu.async_copy(c_v[slot2], out_hbm.at[src(i)], s_o[slot2])

        def wait_out(i, slot2):
            pltpu.make_async_copy(
                c_v[slot2], out_hbm.at[src(i)], s_o[slot2]).wait()

        def compute(slot2):
            f_r, g_r, c_r = f_v[slot2], g_v[slot2], c_v[slot2]

            def vec_body(j, carry):
                base = j * (LANES * UNROLL)
                for u in range(UNROLL):
                    s = pl.ds(base + u * LANES, LANES)
                    c_r[s] = jnp.where(f_r[s] == 1, g_r[s], c_r[s])
                return carry

            lax.fori_loop(0, CHUNK // (LANES * UNROLL), vec_body, 0)

        def step(i, b, do_wait_out_prev, do_next, do_ids2):
            if do_wait_out_prev:
                wait_out(i - 1, (b - 1) % 2)
            if do_next:
                wait_ids(i + 1, (b + 1) % 4)
                start_gather((b + 1) % 4, (b + 1) % 2)
                start_fc(i + 1, (b + 1) % 2)
            if do_ids2:
                start_ids(i + 2, (b + 2) % 4)
            wait_gather(b % 4, b % 2)
            wait_fc(i, b % 2)
            compute(b % 2)
            start_out(i, b % 2)

        start_ids(0, 0)
        start_ids(1, 1)
        wait_ids(0, 0)
        start_gather(0, 0)
        start_fc(0, 0)

        step(0, 0, False, True, True)
        step(1, 1, True, True, True)
        step(2, 2, True, True, True)
        step(3, 3, True, True, True)

        def group_body(g, carry):
            i0 = g * 4
            step(i0 + 0, 0, True, True, True)
            step(i0 + 1, 1, True, True, True)
            step(i0 + 2, 2, True, True, True)
            step(i0 + 3, 3, True, True, True)
            return carry

        lax.fori_loop(1, GROUPS - 1, group_body, 0)

        iL = NCHUNK - 4
        step(iL + 0, 0, True, True, True)
        step(iL + 1, 1, True, True, True)
        step(iL + 2, 2, True, True, False)
        step(iL + 3, 3, True, False, False)
        wait_out(NCHUNK - 1, (NCHUNK - 1) % 2)

    return k(ctx_flat, flag_flat, ids_flat, table)


def kernel(contexts, missing_flag, cell_ids, learning_cell):
    ids = cell_ids.astype(jnp.int32).reshape(-1)
    filled = _sc_fill(
        contexts.reshape(-1), missing_flag.reshape(-1), ids, learning_cell
    )
    return filled.reshape(contexts.shape), learning_cell


# speedup vs baseline: 322.3000x; 1.8548x over previous
import functools

import jax
import jax.numpy as jnp
from jax import lax
from jax.experimental import pallas as pl
from jax.experimental.pallas import tpu as pltpu
from jax.experimental.pallas import tpu_sc as plsc

P, N, W = 26, 16384, 50
E = P * N * W
NUM_WORKERS = 32
PER_WORKER = E // NUM_WORKERS
CHUNK = 12800
NCHUNK = PER_WORKER // CHUNK
GROUPS = NCHUNK // 4
LANES = 16
UNROLL = 4


def _sc_fill(ctx_flat, flag_flat, ids_flat, table):
    mesh = plsc.VectorSubcoreMesh(core_axis_name="c", subcore_axis_name="s")

    @functools.partial(
        pl.kernel,
        mesh=mesh,
        out_type=jax.ShapeDtypeStruct((E,), jnp.float32),
        scratch_types=[pltpu.VMEM((CHUNK,), jnp.int32)] * 4
        + [pltpu.VMEM((CHUNK,), jnp.float32)] * 2
        + [pltpu.VMEM((CHUNK,), jnp.int32)] * 2
        + [pltpu.VMEM((CHUNK,), jnp.float32)] * 2
        + [pltpu.SemaphoreType.DMA] * 12,
    )
    def k(ctx_hbm, flag_hbm, ids_hbm, tab_hbm, out_hbm,
          i0, i1, i2, i3, g0, g1, f0, f1, c0, c1,
          si0, si1, si2, si3, sg0, sg1, sf0, sf1, sc0, sc1, so0, so1):
        idx_v = (i0, i1, i2, i3)
        g_v = (g0, g1)
        f_v = (f0, f1)
        c_v = (c0, c1)
        s_idx = (si0, si1, si2, si3)
        s_g = (sg0, sg1)
        s_f = (sf0, sf1)
        s_c = (sc0, sc1)
        s_o = (so0, so1)
        wid = lax.axis_index("s") * 2 + lax.axis_index("c")
        base_w = wid * PER_WORKER

        def src(i):
            return pl.ds(base_w + i * CHUNK, CHUNK)

        def start_ids(i, slot):
            pltpu.async_copy(ids_hbm.at[src(i)], idx_v[slot], s_idx[slot])

        def wait_ids(i, slot):
            pltpu.make_async_copy(
                ids_hbm.at[src(i)], idx_v[slot], s_idx[slot]).wait()

        def start_gather(slot_i, slot2):
            pltpu.async_copy(tab_hbm.at[idx_v[slot_i]], g_v[slot2], s_g[slot2])

        def wait_gather(slot_i, slot2):
            pltpu.make_async_copy(tab_hbm.at[idx_v[slot_i]], g_v[slot2],
                                  s_g[slot2]).wait()

        def start_fc(i, slot2):
            pltpu.async_copy(flag_hbm.at[src(i)], f_v[slot2], s_f[slot2])
            pltpu.async_copy(ctx_hbm.at[src(i)], c_v[slot2], s_c[slot2])

        def wait_fc(i, slot2):
            pltpu.make_async_copy(
                flag_hbm.at[src(i)], f_v[slot2], s_f[slot2]).wait()
            pltpu.make_async_copy(
                ctx_hbm.at[src(i)], c_v[slot2], s_c[slot2]).wait()

        def start_out(i, slot2):
            pltpu.async_copy(c_v[slot2], out_hbm.at[src(i)], s_o[slot2])

        def wait_out(i, slot2):
            pltpu.make_async_copy(
                c_v[slot2], out_hbm.at[src(i)], s_o[slot2]).wait()

        def compute(slot2):
            f_r, g_r, c_r = f_v[slot2], g_v[slot2], c_v[slot2]

            def vec_body(j, carry):
                base = j * (LANES * UNROLL)
                for u in range(UNROLL):
                    s = pl.ds(base + u * LANES, LANES)
                    c_r[s] = jnp.where(f_r[s] == 1, g_r[s], c_r[s])
                return carry

            lax.fori_loop(0, CHUNK // (LANES * UNROLL), vec_body, 0)

        def step(i, b, do_wait_out_prev, do_next, do_ids2):
            if do_wait_out_prev:
                wait_out(i - 1, (b - 1) % 2)
            if do_next:
                wait_ids(i + 1, (b + 1) % 4)
                start_gather((b + 1) % 4, (b + 1) % 2)
                start_fc(i + 1, (b + 1) % 2)
            if do_ids2:
                start_ids(i + 2, (b + 2) % 4)
            wait_gather(b % 4, b % 2)
            wait_fc(i, b % 2)
            compute(b % 2)
            start_out(i, b % 2)

        start_ids(0, 0)
        start_ids(1, 1)
        wait_ids(0, 0)
        start_gather(0, 0)
        start_fc(0, 0)

        step(0, 0, False, True, True)
        step(1, 1, True, True, True)
        step(2, 2, True, True, True)
        step(3, 3, True, True, True)

        def group_body(g, carry):
            i0 = g * 4
            step(i0 + 0, 0, True, True, True)
            step(i0 + 1, 1, True, True, True)
            step(i0 + 2, 2, True, True, True)
            step(i0 + 3, 3, True, True, True)
            return carry

        lax.fori_loop(1, GROUPS - 1, group_body, 0)

        iL = NCHUNK - 4
        step(iL + 0, 0, True, True, True)
        step(iL + 1, 1, True, True, True)
        step(iL + 2, 2, True, True, False)
        step(iL + 3, 3, True, False, False)
        wait_out(NCHUNK - 1, (NCHUNK - 1) % 2)

    return k(ctx_flat, flag_flat, ids_flat, table)


def kernel(contexts, missing_flag, cell_ids, learning_cell):
    def flat(x):
        return jnp.transpose(x, (0, 2, 1)).reshape(-1)

    ids = flat(cell_ids.astype(jnp.int32))
    filled = _sc_fill(flat(contexts), flat(missing_flag), ids, learning_cell)
    filled = jnp.transpose(filled.reshape(P, W, N), (0, 2, 1))
    return filled, learning_cell
